# Initial kernel scaffold; baseline (speedup 1.0000x reference)
import jax
import jax.numpy as jnp
from jax.experimental import pallas as pl


def kernel(table, anchor, positive, negative):
    raise NotImplementedError("write your pallas kernel here")


# speedup vs baseline: 3.2723x; 3.2723x over previous
import functools

import jax
import jax.numpy as jnp
from jax import lax
from jax.experimental import pallas as pl
from jax.experimental.pallas import tpu as pltpu
from jax.experimental.pallas import tpu_sc as plsc

VOCAB = 100000
D = 128
B = 4096
NNEG = 50

NC = 2
NS = 16
NW = NC * NS

BA = B // NW
BN = B * NNEG // NW
CH = 128
NCH = BN // CH

_mesh = plsc.VectorSubcoreMesh(core_axis_name="c", subcore_axis_name="s")


@functools.partial(
    pl.kernel,
    out_type=(
        jax.ShapeDtypeStruct((B, D), jnp.float32),
        jax.ShapeDtypeStruct((B, D), jnp.float32),
        jax.ShapeDtypeStruct((B * NNEG, D), jnp.float32),
    ),
    mesh=_mesh,
    scratch_types=[
        pltpu.VMEM((BA,), jnp.int32),
        pltpu.VMEM((BA,), jnp.int32),
        pltpu.VMEM((BN,), jnp.int32),
        pltpu.VMEM((CH, D), jnp.float32),
        pltpu.VMEM((CH, D), jnp.float32),
        pltpu.SemaphoreType.DMA,
        pltpu.SemaphoreType.DMA,
    ],
)
def _gather3(tab, anc, pos, neg, outa, outp, outn,
             idx_a, idx_p, idx_n, buf0, buf1, sem0, sem1):
    wid = lax.axis_index("s") * NC + lax.axis_index("c")
    abase = wid * BA
    nbase = wid * BN

    pltpu.sync_copy(anc.at[pl.ds(abase, BA)], idx_a)
    pltpu.sync_copy(pos.at[pl.ds(abase, BA)], idx_p)
    pltpu.sync_copy(neg.at[pl.ds(nbase, BN)], idx_n)

    pltpu.async_copy(tab.at[idx_a], buf0, sem0)
    pltpu.async_copy(tab.at[idx_p], buf1, sem1)
    pltpu.make_async_copy(tab.at[idx_a], buf0, sem0).wait()
    pltpu.sync_copy(buf0, outa.at[pl.ds(abase, BA)])
    pltpu.make_async_copy(tab.at[idx_p], buf1, sem1).wait()
    pltpu.sync_copy(buf1, outp.at[pl.ds(abase, BA)])

    def gath(c, buf, sem):
        pltpu.async_copy(tab.at[idx_n.at[pl.ds(c * CH, CH)]], buf, sem)

    def wait_scat(c, buf, sem):
        pltpu.make_async_copy(tab.at[idx_n.at[pl.ds(c * CH, CH)]], buf, sem).wait()
        pltpu.sync_copy(buf, outn.at[pl.ds(nbase + c * CH, CH)])

    gath(0, buf0, sem0)
    gath(1, buf1, sem1)

    def body(i, _):
        c = 2 * i
        wait_scat(c, buf0, sem0)
        gath(c + 2, buf0, sem0)
        wait_scat(c + 1, buf1, sem1)
        gath(c + 3, buf1, sem1)
        return _

    lax.fori_loop(0, NCH // 2 - 1, body, None)
    wait_scat(NCH - 2, buf0, sem0)
    wait_scat(NCH - 1, buf1, sem1)


def kernel(table, anchor, positive, negative):
    anc = anchor.astype(jnp.int32)
    pos = positive.astype(jnp.int32)
    neg = negative.astype(jnp.int32).reshape(-1)
    outa, outp, outn = _gather3(table, anc, pos, neg)
    return outa, outp, outn.reshape(B, NNEG, D)


# speedup vs baseline: 3.3058x; 1.0102x over previous
import functools

import jax
import jax.numpy as jnp
from jax import lax
from jax.experimental import pallas as pl
from jax.experimental.pallas import tpu as pltpu
from jax.experimental.pallas import tpu_sc as plsc

VOCAB = 100000
D = 128
B = 4096
NNEG = 50

NC = 2
NS = 16
NW = NC * NS

BA = B // NW
BN = B * NNEG // NW
CH = 128
NCH = BN // CH

NBUF = 6
DEPTH = 3
T = 2 + NCH

_mesh = plsc.VectorSubcoreMesh(core_axis_name="c", subcore_axis_name="s")


@functools.partial(
    pl.kernel,
    out_type=(
        jax.ShapeDtypeStruct((B, D), jnp.float32),
        jax.ShapeDtypeStruct((B, D), jnp.float32),
        jax.ShapeDtypeStruct((B * NNEG, D), jnp.float32),
    ),
    mesh=_mesh,
    scratch_types=(
        [pltpu.VMEM((BA,), jnp.int32),
         pltpu.VMEM((BA,), jnp.int32),
         pltpu.VMEM((BN,), jnp.int32)]
        + [pltpu.VMEM((CH, D), jnp.float32)] * NBUF
        + [pltpu.SemaphoreType.DMA] * (2 * NBUF)
    ),
)
def _gather3(tab, anc, pos, neg, outa, outp, outn, idx_a, idx_p, idx_n, *rest):
    bufs = rest[:NBUF]
    gs = rest[NBUF:2 * NBUF]
    ss = rest[2 * NBUF:]

    wid = lax.axis_index("s") * NC + lax.axis_index("c")
    abase = wid * BA
    nbase = wid * BN

    pltpu.sync_copy(anc.at[pl.ds(abase, BA)], idx_a)
    pltpu.sync_copy(pos.at[pl.ds(abase, BA)], idx_p)
    pltpu.sync_copy(neg.at[pl.ds(nbase, BN)], idx_n)

    def idx_src(t):
        if isinstance(t, int) and t == 0:
            return idx_a
        if isinstance(t, int) and t == 1:
            return idx_p
        return idx_n.at[pl.ds((t - 2) * CH, CH)]

    def dst(t):
        if isinstance(t, int) and t == 0:
            return outa.at[pl.ds(abase, BA)]
        if isinstance(t, int) and t == 1:
            return outp.at[pl.ds(abase, BA)]
        return outn.at[pl.ds(nbase + (t - 2) * CH, CH)]

    def gath(t, j):
        pltpu.async_copy(tab.at[idx_src(t)], bufs[j], gs[j])

    def wait_g(t, j):
        pltpu.make_async_copy(tab.at[idx_src(t)], bufs[j], gs[j]).wait()

    def scat(t, j):
        pltpu.async_copy(bufs[j], dst(t), ss[j])

    def wait_s(t, j):
        pltpu.make_async_copy(bufs[j], dst(t), ss[j]).wait()

    def step(t, j, do_swait, do_gath):
        if do_gath:
            j2 = (t + DEPTH) % NBUF
            if do_swait:
                wait_s(t + DEPTH - NBUF, j2)
            gath(t + DEPTH, j2)
        wait_g(t, j)
        scat(t, j)

    for t in range(DEPTH):
        gath(t, t % NBUF)

    t0 = NBUF - DEPTH
    for t in range(t0):
        step(t, t % NBUF, do_swait=False, do_gath=True)

    n_iter = (T - DEPTH - t0) // NBUF
    t_mid_end = t0 + n_iter * NBUF

    def body(i, _):
        base = t0 + i * NBUF
        for k in range(NBUF):
            t = base + k
            j = (t0 + k) % NBUF
            j2 = (t0 + k + DEPTH) % NBUF
            tw = lax.max(t + DEPTH - NBUF, 2)
            wait_s(tw, j2)
            gath(t + DEPTH, j2)
            wait_g(t, j)
            scat(t, j)
        return _

    lax.fori_loop(0, n_iter, body, None)

    for t in range(t_mid_end, T):
        step(t, t % NBUF, do_swait=True, do_gath=(t + DEPTH <= T - 1))

    for t in range(T - NBUF, T):
        wait_s(t, t % NBUF)


def kernel(table, anchor, positive, negative):
    anc = anchor.astype(jnp.int32)
    pos = positive.astype(jnp.int32)
    neg = negative.astype(jnp.int32).reshape(-1)
    outa, outp, outn = _gather3(table, anc, pos, neg)
    return outa, outp, outn.reshape(B, NNEG, D)


# speedup vs baseline: 10.0288x; 3.0337x over previous
import functools

import jax
import jax.numpy as jnp
from jax import lax
from jax.experimental import pallas as pl
from jax.experimental.pallas import tpu as pltpu
from jax.experimental.pallas import tpu_sc as plsc

VOCAB = 100000
D = 128
B = 4096
NNEG = 50

NC = 2
NS = 16
NW = NC * NS

BA = B // NW
BN = B * NNEG // NW
CH = 128
NCH = BN // CH

NBUF = 6
DEPTH = 3
T = 2 + NCH

_mesh = plsc.VectorSubcoreMesh(core_axis_name="c", subcore_axis_name="s")


@functools.partial(
    pl.kernel,
    out_type=(
        jax.ShapeDtypeStruct((B, D), jnp.float32),
        jax.ShapeDtypeStruct((B, D), jnp.float32),
        jax.ShapeDtypeStruct((B * NNEG, D), jnp.float32),
    ),
    mesh=_mesh,
    scratch_types=(
        [pltpu.VMEM((BA,), jnp.int32),
         pltpu.VMEM((BA,), jnp.int32),
         pltpu.VMEM((BN,), jnp.int32)]
        + [pltpu.VMEM((CH, D), jnp.float32)] * NBUF
        + [pltpu.SemaphoreType.DMA] * (2 * NBUF)
    ),
)
def _gather3(tab, anc, pos, neg, outa, outp, outn, idx_a, idx_p, idx_n, *rest):
    bufs = rest[:NBUF]
    gs = rest[NBUF:2 * NBUF]
    ss = rest[2 * NBUF:]

    wid = lax.axis_index("s") * NC + lax.axis_index("c")
    abase = wid * BA
    nbase = wid * BN

    pltpu.sync_copy(anc.at[pl.ds(abase, BA)], idx_a)
    pltpu.sync_copy(pos.at[pl.ds(abase, BA)], idx_p)
    pltpu.sync_copy(neg.at[pl.ds(nbase, BN)], idx_n)

    def idx_src(t):
        if isinstance(t, int) and t == 0:
            return idx_a
        if isinstance(t, int) and t == 1:
            return idx_p
        return idx_n.at[pl.ds((t - 2) * CH, CH)]

    def dst(t):
        if isinstance(t, int) and t == 0:
            return outa.at[pl.ds(abase, BA)]
        if isinstance(t, int) and t == 1:
            return outp.at[pl.ds(abase, BA)]
        return outn.at[pl.ds(nbase + (t - 2) * CH, CH)]

    def gath(t, j):
        pltpu.async_copy(tab.at[idx_src(t)], bufs[j], gs[j])

    def wait_g(t, j):
        pltpu.make_async_copy(tab.at[idx_src(t)], bufs[j], gs[j]).wait()

    def scat(t, j):
        pltpu.async_copy(bufs[j], dst(t), ss[j])

    def wait_s(t, j):
        pltpu.make_async_copy(bufs[j], dst(t), ss[j]).wait()

    def step(t, j, do_swait, do_gath):
        if do_gath:
            j2 = (t + DEPTH) % NBUF
            if do_swait:
                wait_s(t + DEPTH - NBUF, j2)
            gath(t + DEPTH, j2)
        wait_g(t, j)
        scat(t, j)

    for t in range(DEPTH):
        gath(t, t % NBUF)

    t0 = NBUF - DEPTH
    for t in range(t0):
        step(t, t % NBUF, do_swait=False, do_gath=True)

    n_iter = (T - DEPTH - t0) // NBUF
    t_mid_end = t0 + n_iter * NBUF

    def body(i, _):
        base = t0 + i * NBUF
        for k in range(NBUF):
            t = base + k
            j = (t0 + k) % NBUF
            j2 = (t0 + k + DEPTH) % NBUF
            tw = lax.max(t + DEPTH - NBUF, 2)
            wait_s(tw, j2)
            gath(t + DEPTH, j2)
            wait_g(t, j)
            scat(t, j)
        return _

    lax.fori_loop(0, n_iter, body, None)

    for t in range(t_mid_end, T):
        step(t, t % NBUF, do_swait=True, do_gath=(t + DEPTH <= T - 1))

    for t in range(T - NBUF, T):
        wait_s(t, t % NBUF)


def kernel(table, anchor, positive, negative):
    anc = anchor.astype(jnp.int32)
    pos = positive.astype(jnp.int32)
    neg = negative.astype(jnp.int32).T.reshape(-1)
    outa, outp, outn = _gather3(table, anc, pos, neg)
    return outa, outp, outn.reshape(NNEG, B, D).transpose(1, 0, 2)
